# baseline (speedup 1.0000x reference)
import jax
import jax.numpy as jnp
from jax import lax
from jax.experimental import pallas as pl
from jax.experimental.pallas import tpu as pltpu
from jax.experimental.pallas import tpu_sc as plsc

V = 1000000
D = 100
DP = 128
B = 16384
L = 200

NC = 2
NS = 16
NW = NC * NS

N = B * L
PER_W = N // NW
CHUNK = 128
SUB = 128
NBUF = 4
STEPS = PER_W // CHUNK
UNROLL = 16
IDX_T = N // (8 * SUB)

ADD_BLOCK = 512
ADD_GRID = -(-V // ADD_BLOCK)


def _add_body(aT_ref, bT_ref, o_ref):
    s = aT_ref[...] + bT_ref[...]
    o_ref[:, :D] = jnp.swapaxes(s, 0, 1)


def _combine(glove_table, weight):
    return pl.pallas_call(
        _add_body,
        out_shape=jax.ShapeDtypeStruct((V, DP), jnp.float32),
        grid=(ADD_GRID,),
        in_specs=[
            pl.BlockSpec((D, ADD_BLOCK), lambda i: (0, i)),
            pl.BlockSpec((D, ADD_BLOCK), lambda i: (0, i)),
        ],
        out_specs=pl.BlockSpec((ADD_BLOCK, DP), lambda i: (i, 0)),
    )(glove_table.T, weight.T)


def _gather_body(idx_hbm, tbl_hbm, out_hbm, idx_v, rows_v,
                 sg0, sg1, sg2, sg3, so0, so1, so2, so3):
    wid = lax.axis_index("s") * NC + lax.axis_index("c")
    tiles_per_w = PER_W // (8 * SUB)
    sg = (sg0, sg1, sg2, sg3)
    so = (so0, so1, so2, so3)

    def fire_gather(c_dyn, itb, row, b):
        pltpu.async_copy(
            tbl_hbm.at[idx_v.at[itb, row]], rows_v.at[b], sg[b]
        )

    def fire_out(c_dyn, b):
        out_row = wid * PER_W + c_dyn * CHUNK
        pltpu.async_copy(rows_v.at[b], out_hbm.at[pl.ds(out_row, CHUNK)], so[b])

    def wait_gather(b):
        pltpu.make_async_copy(
            out_hbm.at[pl.ds(0, CHUNK)], rows_v.at[b], sg[b]
        ).wait()

    def wait_out(b):
        pltpu.make_async_copy(
            rows_v.at[b], out_hbm.at[pl.ds(0, CHUNK)], so[b]
        ).wait()

    pltpu.sync_copy(idx_hbm.at[wid * tiles_per_w], idx_v.at[0])
    fire_gather(0, 0, 0, 0)

    def body(k, carry):
        c0 = k * UNROLL
        for j in range(UNROLL):
            c = c0 + j
            b = j % NBUF
            itb = (j // 8) % 2
            if j % 8 == 0:
                pltpu.sync_copy(
                    idx_hbm.at[wid * tiles_per_w + c // 8], idx_v.at[itb]
                )
            @pl.when(c >= NBUF)
            def _(b=b):
                wait_out(b)

            if j == 0:
                @pl.when(k > 0)
                def _(itb=itb, b=b):
                    fire_gather(c, itb, j % 8, b)
            else:
                fire_gather(c, itb, j % 8, b)

            bp = (j - 1) % NBUF

            @pl.when(c > 0)
            def _(c=c, bp=bp):
                wait_gather(bp)
                fire_out(c - 1, bp)

        return carry

    lax.fori_loop(0, STEPS // UNROLL, body, 0)
    last_b = (STEPS - 1) % NBUF
    wait_gather(last_b)
    fire_out(STEPS - 1, last_b)
    for b in range(NBUF):
        wait_out(b)


def kernel(indices, glove_table, weight):
    combined = _combine(glove_table, weight)
    idx3d = indices.reshape(IDX_T, 8, SUB).astype(jnp.int32)
    out = pl.kernel(
        _gather_body,
        out_type=jax.ShapeDtypeStruct((N, DP), jnp.float32),
        mesh=plsc.VectorSubcoreMesh(core_axis_name="c", subcore_axis_name="s"),
        compiler_params=pltpu.CompilerParams(use_tc_tiling_on_sc=True),
        scratch_types=[
            pltpu.VMEM((2, 8, SUB), jnp.int32),
            pltpu.VMEM((NBUF, CHUNK, DP), jnp.float32),
        ] + [pltpu.SemaphoreType.DMA] * 8,
    )(idx3d, combined)
    return out[:, :D].reshape(B, L, D)


# speedup vs baseline: 1.2547x; 1.2547x over previous
import jax
import jax.numpy as jnp
from jax import lax
from jax.experimental import pallas as pl
from jax.experimental.pallas import tpu as pltpu
from jax.experimental.pallas import tpu_sc as plsc

V = 1000000
D = 100
DP = 128
B = 16384
L = 200

NC = 2
NS = 16
NW = NC * NS

N = B * L
PER_W = N // NW
CHUNK = 128
SUB = 128
NBUF = 4
STEPS = PER_W // CHUNK
UNROLL = 16
IDX_T = N // (8 * SUB)

ADD_BLOCK = 2048
ADD_GRID = -(-V // ADD_BLOCK)


def _add_body(aT_ref, bT_ref, o_ref):
    s = aT_ref[...] + bT_ref[...]
    o_ref[:, :D] = jnp.swapaxes(s, 0, 1)


def _combine(glove_table, weight):
    return pl.pallas_call(
        _add_body,
        out_shape=jax.ShapeDtypeStruct((V, DP), jnp.float32),
        grid=(ADD_GRID,),
        in_specs=[
            pl.BlockSpec((D, ADD_BLOCK), lambda i: (0, i)),
            pl.BlockSpec((D, ADD_BLOCK), lambda i: (0, i)),
        ],
        out_specs=pl.BlockSpec((ADD_BLOCK, DP), lambda i: (i, 0)),
    )(glove_table.T, weight.T)


def _gather_body(idx_hbm, tbl_hbm, out_hbm, idx_v, rows_v,
                 sg0, sg1, sg2, sg3, so0, so1, so2, so3):
    wid = lax.axis_index("s") * NC + lax.axis_index("c")
    tiles_per_w = PER_W // (8 * SUB)
    sg = (sg0, sg1, sg2, sg3)
    so = (so0, so1, so2, so3)

    def fire_gather(c_dyn, itb, row, b):
        pltpu.async_copy(
            tbl_hbm.at[idx_v.at[itb, row]], rows_v.at[b], sg[b]
        )

    def fire_out(c_dyn, b):
        out_row = wid * PER_W + c_dyn * CHUNK
        pltpu.async_copy(rows_v.at[b], out_hbm.at[pl.ds(out_row, CHUNK)], so[b])

    def wait_gather(b):
        pltpu.make_async_copy(
            out_hbm.at[pl.ds(0, CHUNK)], rows_v.at[b], sg[b]
        ).wait()

    def wait_out(b):
        pltpu.make_async_copy(
            rows_v.at[b], out_hbm.at[pl.ds(0, CHUNK)], so[b]
        ).wait()

    pltpu.sync_copy(idx_hbm.at[wid * tiles_per_w], idx_v.at[0])
    fire_gather(0, 0, 0, 0)

    def body(k, carry):
        c0 = k * UNROLL
        for j in range(UNROLL):
            c = c0 + j
            b = j % NBUF
            itb = (j // 8) % 2
            if j % 8 == 0:
                pltpu.sync_copy(
                    idx_hbm.at[wid * tiles_per_w + c // 8], idx_v.at[itb]
                )
            @pl.when(c >= NBUF)
            def _(b=b):
                wait_out(b)

            if j == 0:
                @pl.when(k > 0)
                def _(itb=itb, b=b):
                    fire_gather(c, itb, j % 8, b)
            else:
                fire_gather(c, itb, j % 8, b)

            bp = (j - 1) % NBUF

            @pl.when(c > 0)
            def _(c=c, bp=bp):
                wait_gather(bp)
                fire_out(c - 1, bp)

        return carry

    lax.fori_loop(0, STEPS // UNROLL, body, 0)
    last_b = (STEPS - 1) % NBUF
    wait_gather(last_b)
    fire_out(STEPS - 1, last_b)
    for b in range(NBUF):
        wait_out(b)


def kernel(indices, glove_table, weight):
    combined = _combine(glove_table, weight)
    idx3d = indices.reshape(IDX_T, 8, SUB).astype(jnp.int32)
    out = pl.kernel(
        _gather_body,
        out_type=jax.ShapeDtypeStruct((N, DP), jnp.float32),
        mesh=plsc.VectorSubcoreMesh(core_axis_name="c", subcore_axis_name="s"),
        compiler_params=pltpu.CompilerParams(use_tc_tiling_on_sc=True),
        scratch_types=[
            pltpu.VMEM((2, 8, SUB), jnp.int32),
            pltpu.VMEM((NBUF, CHUNK, DP), jnp.float32),
        ] + [pltpu.SemaphoreType.DMA] * 8,
    )(idx3d, combined)
    return out[:, :D].reshape(B, L, D)


# speedup vs baseline: 1.3352x; 1.0642x over previous
import jax
import jax.numpy as jnp
from jax import lax
from jax.experimental import pallas as pl
from jax.experimental.pallas import tpu as pltpu
from jax.experimental.pallas import tpu_sc as plsc

V = 1000000
D = 100
DP = 128
B = 16384
L = 200

NC = 2
NS = 16
NW = NC * NS

N = B * L
PER_W = N // NW
CHUNK = 128
SUB = 128
NBUF = 4
STEPS = PER_W // CHUNK
UNROLL = 16
IDX_T = N // (8 * SUB)

ADD_BLOCK = 8192
ADD_GRID = -(-V // ADD_BLOCK)


def _add_body(aT_ref, bT_ref, o_ref):
    s = aT_ref[...] + bT_ref[...]
    o_ref[:, :D] = jnp.swapaxes(s, 0, 1)


def _combine(glove_table, weight):
    return pl.pallas_call(
        _add_body,
        out_shape=jax.ShapeDtypeStruct((V, DP), jnp.float32),
        grid=(ADD_GRID,),
        in_specs=[
            pl.BlockSpec((D, ADD_BLOCK), lambda i: (0, i)),
            pl.BlockSpec((D, ADD_BLOCK), lambda i: (0, i)),
        ],
        out_specs=pl.BlockSpec((ADD_BLOCK, DP), lambda i: (i, 0)),
    )(glove_table.T, weight.T)


def _gather_body(idx_hbm, tbl_hbm, out_hbm, idx_v, rows_v,
                 sg0, sg1, sg2, sg3, so0, so1, so2, so3):
    wid = lax.axis_index("s") * NC + lax.axis_index("c")
    tiles_per_w = PER_W // (8 * SUB)
    sg = (sg0, sg1, sg2, sg3)
    so = (so0, so1, so2, so3)

    def fire_gather(c_dyn, itb, row, b):
        pltpu.async_copy(
            tbl_hbm.at[idx_v.at[itb, row]], rows_v.at[b], sg[b]
        )

    def fire_out(c_dyn, b):
        out_row = wid * PER_W + c_dyn * CHUNK
        pltpu.async_copy(rows_v.at[b], out_hbm.at[pl.ds(out_row, CHUNK)], so[b])

    def wait_gather(b):
        pltpu.make_async_copy(
            out_hbm.at[pl.ds(0, CHUNK)], rows_v.at[b], sg[b]
        ).wait()

    def wait_out(b):
        pltpu.make_async_copy(
            rows_v.at[b], out_hbm.at[pl.ds(0, CHUNK)], so[b]
        ).wait()

    pltpu.sync_copy(idx_hbm.at[wid * tiles_per_w], idx_v.at[0])
    fire_gather(0, 0, 0, 0)

    def body(k, carry):
        c0 = k * UNROLL
        for j in range(UNROLL):
            c = c0 + j
            b = j % NBUF
            itb = (j // 8) % 2
            if j % 8 == 0:
                pltpu.sync_copy(
                    idx_hbm.at[wid * tiles_per_w + c // 8], idx_v.at[itb]
                )
            @pl.when(c >= NBUF)
            def _(b=b):
                wait_out(b)

            if j == 0:
                @pl.when(k > 0)
                def _(itb=itb, b=b):
                    fire_gather(c, itb, j % 8, b)
            else:
                fire_gather(c, itb, j % 8, b)

            bp = (j - 1) % NBUF

            @pl.when(c > 0)
            def _(c=c, bp=bp):
                wait_gather(bp)
                fire_out(c - 1, bp)

        return carry

    lax.fori_loop(0, STEPS // UNROLL, body, 0)
    last_b = (STEPS - 1) % NBUF
    wait_gather(last_b)
    fire_out(STEPS - 1, last_b)
    for b in range(NBUF):
        wait_out(b)


def kernel(indices, glove_table, weight):
    combined = _combine(glove_table, weight)
    idx3d = indices.reshape(IDX_T, 8, SUB).astype(jnp.int32)
    out = pl.kernel(
        _gather_body,
        out_type=jax.ShapeDtypeStruct((N, DP), jnp.float32),
        mesh=plsc.VectorSubcoreMesh(core_axis_name="c", subcore_axis_name="s"),
        compiler_params=pltpu.CompilerParams(use_tc_tiling_on_sc=True),
        scratch_types=[
            pltpu.VMEM((2, 8, SUB), jnp.int32),
            pltpu.VMEM((NBUF, CHUNK, DP), jnp.float32),
        ] + [pltpu.SemaphoreType.DMA] * 8,
    )(idx3d, combined)
    return out[:, :D].reshape(B, L, D)


# speedup vs baseline: 1.3391x; 1.0030x over previous
import jax
import jax.numpy as jnp
from jax import lax
from jax.experimental import pallas as pl
from jax.experimental.pallas import tpu as pltpu
from jax.experimental.pallas import tpu_sc as plsc

V = 1000000
D = 100
DP = 128
B = 16384
L = 200

NC = 2
NS = 16
NW = NC * NS

N = B * L
PER_W = N // NW
CHUNK = 128
SUB = 128
NBUF = 4
STEPS = PER_W // CHUNK
UNROLL = 16
IDX_T = N // (8 * SUB)

ADD_BLOCK = 16384
ADD_GRID = -(-V // ADD_BLOCK)


def _add_body(aT_ref, bT_ref, o_ref):
    s = aT_ref[...] + bT_ref[...]
    o_ref[:, :D] = jnp.swapaxes(s, 0, 1)


def _combine(glove_table, weight):
    return pl.pallas_call(
        _add_body,
        out_shape=jax.ShapeDtypeStruct((V, DP), jnp.float32),
        grid=(ADD_GRID,),
        in_specs=[
            pl.BlockSpec((D, ADD_BLOCK), lambda i: (0, i)),
            pl.BlockSpec((D, ADD_BLOCK), lambda i: (0, i)),
        ],
        out_specs=pl.BlockSpec((ADD_BLOCK, DP), lambda i: (i, 0)),
    )(glove_table.T, weight.T)


def _gather_body(idx_hbm, tbl_hbm, out_hbm, idx_v, rows_v,
                 sg0, sg1, sg2, sg3, so0, so1, so2, so3):
    wid = lax.axis_index("s") * NC + lax.axis_index("c")
    tiles_per_w = PER_W // (8 * SUB)
    sg = (sg0, sg1, sg2, sg3)
    so = (so0, so1, so2, so3)

    def fire_gather(c_dyn, itb, row, b):
        pltpu.async_copy(
            tbl_hbm.at[idx_v.at[itb, row]], rows_v.at[b], sg[b]
        )

    def fire_out(c_dyn, b):
        out_row = wid * PER_W + c_dyn * CHUNK
        pltpu.async_copy(rows_v.at[b], out_hbm.at[pl.ds(out_row, CHUNK)], so[b])

    def wait_gather(b):
        pltpu.make_async_copy(
            out_hbm.at[pl.ds(0, CHUNK)], rows_v.at[b], sg[b]
        ).wait()

    def wait_out(b):
        pltpu.make_async_copy(
            rows_v.at[b], out_hbm.at[pl.ds(0, CHUNK)], so[b]
        ).wait()

    pltpu.sync_copy(idx_hbm.at[wid * tiles_per_w], idx_v.at[0])
    fire_gather(0, 0, 0, 0)

    def body(k, carry):
        c0 = k * UNROLL
        for j in range(UNROLL):
            c = c0 + j
            b = j % NBUF
            itb = (j // 8) % 2
            if j % 8 == 0:
                pltpu.sync_copy(
                    idx_hbm.at[wid * tiles_per_w + c // 8], idx_v.at[itb]
                )
            @pl.when(c >= NBUF)
            def _(b=b):
                wait_out(b)

            if j == 0:
                @pl.when(k > 0)
                def _(itb=itb, b=b):
                    fire_gather(c, itb, j % 8, b)
            else:
                fire_gather(c, itb, j % 8, b)

            bp = (j - 1) % NBUF

            @pl.when(c > 0)
            def _(c=c, bp=bp):
                wait_gather(bp)
                fire_out(c - 1, bp)

        return carry

    lax.fori_loop(0, STEPS // UNROLL, body, 0)
    last_b = (STEPS - 1) % NBUF
    wait_gather(last_b)
    fire_out(STEPS - 1, last_b)
    for b in range(NBUF):
        wait_out(b)


def kernel(indices, glove_table, weight):
    combined = _combine(glove_table, weight)
    idx3d = indices.reshape(IDX_T, 8, SUB).astype(jnp.int32)
    out = pl.kernel(
        _gather_body,
        out_type=jax.ShapeDtypeStruct((N, DP), jnp.float32),
        mesh=plsc.VectorSubcoreMesh(core_axis_name="c", subcore_axis_name="s"),
        compiler_params=pltpu.CompilerParams(use_tc_tiling_on_sc=True),
        scratch_types=[
            pltpu.VMEM((2, 8, SUB), jnp.int32),
            pltpu.VMEM((NBUF, CHUNK, DP), jnp.float32),
        ] + [pltpu.SemaphoreType.DMA] * 8,
    )(idx3d, combined)
    return out[:, :D].reshape(B, L, D)
